# baseline (speedup 1.0000x reference)
import functools

import jax
import jax.numpy as jnp
from jax import lax
from jax.experimental import pallas as pl
from jax.experimental.pallas import tpu as pltpu
from jax.experimental.pallas import tpu_sc as plsc

N = 100000
D = 128
PICK = 12
KPAD = 16
BN = 20000
GRID = N // BN
NEG = -1e30
IMAX = 2147483647
_DOT_NT = (((1,), (1,)), ((), ()))
_DOT_NN = (((1,), (0,)), ((), ()))


def _sc_gather_rows(embed, idx16):
    mesh = plsc.VectorSubcoreMesh(core_axis_name="c", subcore_axis_name="s")

    @functools.partial(
        pl.kernel,
        out_type=jax.ShapeDtypeStruct((KPAD, D), jnp.float32),
        mesh=mesh,
        scratch_types=[
            pltpu.VMEM((KPAD,), jnp.int32),
            pltpu.VMEM((KPAD, D), jnp.float32),
            pltpu.SemaphoreType.DMA,
        ],
    )
    def gather_kernel(embed_hbm, idx_hbm, out_hbm, idx_v, rows_v, sem):
        c = lax.axis_index("c")
        s = lax.axis_index("s")

        @pl.when(jnp.logical_and(c == 0, s == 0))
        def _():
            pltpu.sync_copy(idx_hbm, idx_v)
            pltpu.async_copy(embed_hbm.at[idx_v], rows_v, sem).wait()
            pltpu.sync_copy(rows_v, out_hbm)

    return gather_kernel(embed, idx16)


def _fused_passes(embed, rows_s, Wq, Wk, bq_col, bq_row, bk_row, bk_col):
    def body(embed_any, rows_ref, wq_ref, wk_ref, bqc_ref, bqr_ref,
             bkr_ref, bkc_ref, out_ref, ebb_ref, land_ref, rows2_ref,
             sems, sem_row):
        def chunk_copy(j):
            return pltpu.make_async_copy(
                embed_any.at[pl.ds(j * BN, BN), :],
                land_ref.at[j % 2],
                sems.at[j % 2],
            )

        chunk_copy(0).start()
        if GRID > 1:
            chunk_copy(1).start()
        rows2_ref[...] = jnp.zeros((KPAD, D), jnp.float32)

        nk = lax.dot_general(rows_ref[...], wk_ref[...], _DOT_NT,
                             preferred_element_type=jnp.float32) + bkr_ref[...]
        qa = lax.dot_general(nk, wq_ref[...], _DOT_NN,
                             preferred_element_type=jnp.float32)
        cc = lax.dot_general(nk, bqc_ref[...], _DOT_NN,
                             preferred_element_type=jnp.float32)
        rid = lax.broadcasted_iota(jnp.int32, (KPAD, 1), 0)
        cc = jnp.where(rid >= PICK, NEG, cc)

        lane16 = lax.broadcasted_iota(jnp.int32, (1, KPAD), 1)

        candvs, candis = [], []
        for j in range(GRID):
            chunk_copy(j).wait()
            blk = land_ref[j % 2]
            ebb_ref[j * BN:(j + 1) * BN, :] = blk.astype(jnp.bfloat16)
            st = lax.dot_general(qa, blk, _DOT_NT,
                                 preferred_element_type=jnp.float32)
            mvb = jnp.max(st + cc, axis=0, keepdims=True)
            if j + 2 < GRID:
                chunk_copy(j + 2).start()
            gidx = lax.broadcasted_iota(jnp.int32, (1, BN), 1) + j * BN
            cv = jnp.full((1, KPAD), NEG, jnp.float32)
            ci = jnp.full((1, KPAD), IMAX, jnp.int32)
            for t in range(PICK):
                m = jnp.max(mvb, axis=1, keepdims=True)
                sel = jnp.min(jnp.where(mvb >= m, gidx, IMAX),
                              axis=1, keepdims=True)
                mvb = jnp.where(gidx == sel, NEG, mvb)
                cv = jnp.where(lane16 == t, m, cv)
                ci = jnp.where(lane16 == t, sel, ci)
            candvs.append(cv)
            candis.append(ci)

        cV = jnp.concatenate(candvs, axis=1)
        cI = jnp.concatenate(candis, axis=1)
        copies = []
        for t in range(PICK):
            m = jnp.max(cV)
            sel = jnp.min(jnp.where(cV >= m, cI, IMAX))
            cV = jnp.where((cV >= m) & (cI == sel), NEG, cV)
            cp = pltpu.make_async_copy(
                embed_any.at[pl.ds(sel, 1), :],
                rows2_ref.at[pl.ds(t, 1), :],
                sem_row,
            )
            cp.start()
            copies.append(cp)
        for cp in copies:
            cp.wait()

        qr = lax.dot_general(rows2_ref[...], wq_ref[...], _DOT_NT,
                             preferred_element_type=jnp.float32) + bqr_ref[...]
        bb = lax.dot_general(qr, wk_ref[...], _DOT_NN,
                             preferred_element_type=jnp.float32)
        dd = lax.dot_general(qr, bkc_ref[...], _DOT_NN,
                             preferred_element_type=jnp.float32)
        dd = jnp.where(rid >= PICK, NEG, dd)
        bb16 = bb.astype(jnp.bfloat16)

        acc = jnp.zeros((1, D), jnp.float32)
        for j in range(GRID):
            blkb = ebb_ref[j * BN:(j + 1) * BN, :]
            tt = lax.dot_general(bb16, blkb, _DOT_NT,
                                 preferred_element_type=jnp.float32)
            p = jnp.max(tt + dd, axis=0, keepdims=True)
            acc = acc + lax.dot_general(p.astype(jnp.bfloat16), blkb,
                                        _DOT_NN,
                                        preferred_element_type=jnp.float32)
        out_ref[...] = acc

    return pl.pallas_call(
        body,
        in_specs=[
            pl.BlockSpec(memory_space=pl.ANY),
            pl.BlockSpec(memory_space=pltpu.VMEM),
            pl.BlockSpec(memory_space=pltpu.VMEM),
            pl.BlockSpec(memory_space=pltpu.VMEM),
            pl.BlockSpec(memory_space=pltpu.VMEM),
            pl.BlockSpec(memory_space=pltpu.VMEM),
            pl.BlockSpec(memory_space=pltpu.VMEM),
            pl.BlockSpec(memory_space=pltpu.VMEM),
        ],
        out_specs=pl.BlockSpec(memory_space=pltpu.VMEM),
        out_shape=jax.ShapeDtypeStruct((1, D), jnp.float32),
        scratch_shapes=[
            pltpu.VMEM((N, D), jnp.bfloat16),
            pltpu.VMEM((2, BN, D), jnp.float32),
            pltpu.VMEM((KPAD, D), jnp.float32),
            pltpu.SemaphoreType.DMA((2,)),
            pltpu.SemaphoreType.DMA,
        ],
    )(embed, rows_s, Wq, Wk, bq_col, bq_row, bk_row, bk_col)


def kernel(embed_matrix, Wq, bq, Wk, bk, sample_indices):
    idx16 = jnp.concatenate(
        [sample_indices.astype(jnp.int32),
         jnp.zeros((KPAD - PICK,), jnp.int32)])
    rows_s = _sc_gather_rows(embed_matrix, idx16)
    return _fused_passes(embed_matrix, rows_s, Wq, Wk,
                         bq.reshape(D, 1), bq.reshape(1, D),
                         bk.reshape(1, D), bk.reshape(D, 1))
